# Initial kernel scaffold; baseline (speedup 1.0000x reference)
import functools

import jax
import jax.numpy as jnp
from jax import lax
from jax.experimental import pallas as pl
from jax.experimental.pallas import tpu as pltpu
from jax.experimental.pallas import tpu_sc as plsc

_NC = 2
_NS = 16
_NW = _NC * _NS
_CHUNK = 128
_DEGW = 16


def _deg_kernel_body(dst_hbm, zeros_hbm, ones_hbm, degp_hbm, ones_v, idx_v,
                     acc_shared, *, n_nodes, n_chunks):
    cid = lax.axis_index("c")
    sid = lax.axis_index("s")
    wid = sid * _NC + cid
    rows_per_sub = n_nodes // _NS
    row0 = sid * rows_per_sub
    pltpu.sync_copy(zeros_hbm.at[pl.ds(row0, rows_per_sub)],
                    acc_shared.at[pl.ds(row0, rows_per_sub)])
    pltpu.sync_copy(ones_hbm, ones_v)
    plsc.subcore_barrier()
    n_iters = (n_chunks + _NW - 1) // _NW

    def step(i, carry):
        c = wid + i * _NW

        @pl.when(c < n_chunks)
        def _():
            pltpu.sync_copy(dst_hbm.at[pl.ds(c * _CHUNK, _CHUNK)], idx_v)
            pltpu.sync_copy(ones_v, acc_shared.at[idx_v], add=True)

        return carry

    lax.fori_loop(0, n_iters, step, 0)
    plsc.subcore_barrier()
    pltpu.sync_copy(acc_shared.at[pl.ds(row0, rows_per_sub)],
                    degp_hbm.at[cid, pl.ds(row0, rows_per_sub)])


def _scatter_kernel_body(src_hbm, dst_hbm, hs_hbm, zeros_hbm, part_hbm,
                         sidx_v, didx_v, rows_v, acc_shared, *, n_nodes,
                         n_chunks):
    cid = lax.axis_index("c")
    sid = lax.axis_index("s")
    wid = sid * _NC + cid
    rows_per_sub = n_nodes // _NS
    row0 = sid * rows_per_sub
    pltpu.sync_copy(zeros_hbm.at[pl.ds(row0, rows_per_sub)],
                    acc_shared.at[pl.ds(row0, rows_per_sub)])
    plsc.subcore_barrier()
    n_iters = (n_chunks + _NW - 1) // _NW

    def step(i, carry):
        c = wid + i * _NW

        @pl.when(c < n_chunks)
        def _():
            off = c * _CHUNK
            pltpu.sync_copy(src_hbm.at[pl.ds(off, _CHUNK)], sidx_v)
            pltpu.sync_copy(dst_hbm.at[pl.ds(off, _CHUNK)], didx_v)
            pltpu.sync_copy(hs_hbm.at[sidx_v], rows_v)
            pltpu.sync_copy(rows_v, acc_shared.at[didx_v], add=True)

        return carry

    lax.fori_loop(0, n_iters, step, 0)
    plsc.subcore_barrier()
    pltpu.sync_copy(acc_shared.at[pl.ds(row0, rows_per_sub)],
                    part_hbm.at[cid, pl.ds(row0, rows_per_sub)])


def _sc_degree(dst, n_nodes):
    e = dst.shape[0]
    n_chunks = e // _CHUNK
    zeros = jnp.zeros((n_nodes, _DEGW), jnp.float32)
    ones = jnp.ones((_CHUNK, _DEGW), jnp.float32)
    mesh = plsc.VectorSubcoreMesh(core_axis_name="c", subcore_axis_name="s")
    body = functools.partial(_deg_kernel_body, n_nodes=n_nodes,
                             n_chunks=n_chunks)
    return pl.kernel(
        body,
        out_type=jax.ShapeDtypeStruct((_NC, n_nodes, _DEGW), jnp.float32),
        mesh=mesh,
        scratch_types=[
            pltpu.VMEM((_CHUNK, _DEGW), jnp.float32),
            pltpu.VMEM((_CHUNK,), jnp.int32),
            pltpu.VMEM_SHARED((n_nodes, _DEGW), jnp.float32),
        ],
        name="sc_degree",
    )(dst, zeros, ones)


def _sc_scatter(src, dst, hs, n_nodes):
    e = src.shape[0]
    n_chunks = e // _CHUNK
    d = hs.shape[1]
    zeros = jnp.zeros((n_nodes, d), jnp.float32)
    mesh = plsc.VectorSubcoreMesh(core_axis_name="c", subcore_axis_name="s")
    body = functools.partial(_scatter_kernel_body, n_nodes=n_nodes,
                             n_chunks=n_chunks)
    return pl.kernel(
        body,
        out_type=jax.ShapeDtypeStruct((_NC, n_nodes, d), jnp.float32),
        mesh=mesh,
        scratch_types=[
            pltpu.VMEM((_CHUNK,), jnp.int32),
            pltpu.VMEM((_CHUNK,), jnp.int32),
            pltpu.VMEM((_CHUNK, d), jnp.float32),
            pltpu.VMEM_SHARED((n_nodes, d), jnp.float32),
        ],
        name="sc_scatter",
    )(src, dst, hs, zeros)


_ROWS = 2500


def _tc_scale1_body(degp_ref, x_ref, w_ref, hs_ref, dinv_ref):
    deg = degp_ref[0] + degp_ref[1]
    dinv = lax.rsqrt(jnp.maximum(deg[:, :1] + 1.0, 1.0))
    h = jnp.dot(x_ref[...], w_ref[...], preferred_element_type=jnp.float32)
    hs_ref[...] = h * dinv
    dinv_ref[...] = dinv


def _tc_scale1(degp, x, w1, n_nodes):
    d, h = w1.shape
    grid = (n_nodes // _ROWS,)
    return pl.pallas_call(
        _tc_scale1_body,
        grid=grid,
        in_specs=[
            pl.BlockSpec((_NC, _ROWS, _DEGW), lambda i: (0, i, 0)),
            pl.BlockSpec((_ROWS, d), lambda i: (i, 0)),
            pl.BlockSpec((d, h), lambda i: (0, 0)),
        ],
        out_specs=[
            pl.BlockSpec((_ROWS, h), lambda i: (i, 0)),
            pl.BlockSpec((_ROWS, 1), lambda i: (i, 0)),
        ],
        out_shape=[
            jax.ShapeDtypeStruct((n_nodes, h), jnp.float32),
            jax.ShapeDtypeStruct((n_nodes, 1), jnp.float32),
        ],
        name="tc_scale1",
    )(degp, x, w1)


def _tc_layer2_body(p_ref, hs_ref, dinv_ref, b_ref, w_ref, out_ref):
    dinv = dinv_ref[...]
    agg = (p_ref[0] + p_ref[1] + hs_ref[...]) * dinv
    h1 = jnp.maximum(agg + b_ref[...], 0.0)
    out_ref[...] = jnp.dot(h1, w_ref[...],
                           preferred_element_type=jnp.float32) * dinv


def _tc_layer2(p, hs, dinv, b1, w2, n_nodes):
    h, h2 = w2.shape
    grid = (n_nodes // _ROWS,)
    return pl.pallas_call(
        _tc_layer2_body,
        grid=grid,
        in_specs=[
            pl.BlockSpec((_NC, _ROWS, h), lambda i: (0, i, 0)),
            pl.BlockSpec((_ROWS, h), lambda i: (i, 0)),
            pl.BlockSpec((_ROWS, 1), lambda i: (i, 0)),
            pl.BlockSpec((1, h2), lambda i: (0, 0)),
            pl.BlockSpec((h, h2), lambda i: (0, 0)),
        ],
        out_specs=pl.BlockSpec((_ROWS, h2), lambda i: (i, 0)),
        out_shape=jax.ShapeDtypeStruct((n_nodes, h2), jnp.float32),
        name="tc_layer2",
    )(p, hs, dinv, b1.reshape(1, h2), w2)


def _tc_head_body(p_ref, hs_ref, dinv_ref, b_ref, batch_ref, fw1_ref,
                  fb1_ref, fw2_ref, fb2_ref, emb_ref, out_ref,
                  sums_ref, cnt_ref, *, n_groups, n_blocks):
    i = pl.program_id(0)
    agg = (p_ref[0] + p_ref[1] + hs_ref[...]) * dinv_ref[...]
    h2 = jnp.maximum(agg + b_ref[...], 0.0)
    gids = lax.broadcasted_iota(jnp.float32, (h2.shape[0], n_groups), 1)
    oh = (batch_ref[...] == gids).astype(jnp.float32)
    blk_sums = lax.dot_general(oh, h2, (((0,), (0,)), ((), ())),
                               preferred_element_type=jnp.float32)
    blk_cnt = jnp.sum(oh, axis=0)[:, None]

    @pl.when(i == 0)
    def _():
        sums_ref[...] = blk_sums
        cnt_ref[...] = blk_cnt

    @pl.when(i > 0)
    def _():
        sums_ref[...] += blk_sums
        cnt_ref[...] += blk_cnt

    @pl.when(i == n_blocks - 1)
    def _():
        pooled = sums_ref[...] / jnp.maximum(cnt_ref[...], 1.0)
        emb = jnp.maximum(
            jnp.dot(pooled, fw1_ref[...],
                    preferred_element_type=jnp.float32) + fb1_ref[...], 0.0)
        emb_ref[...] = emb
        out_ref[...] = jnp.dot(emb, fw2_ref[...],
                               preferred_element_type=jnp.float32) + fb2_ref[...]


def _tc_head(p, hs, dinv, b2, batchf, fw1, fb1, fw2, fb2, n_nodes, n_groups):
    h = hs.shape[1]
    emb_d = fw1.shape[1]
    out_d = fw2.shape[1]
    n_blocks = n_nodes // _ROWS
    body = functools.partial(_tc_head_body, n_groups=n_groups,
                             n_blocks=n_blocks)
    return pl.pallas_call(
        body,
        grid=(n_blocks,),
        in_specs=[
            pl.BlockSpec((_NC, _ROWS, h), lambda i: (0, i, 0)),
            pl.BlockSpec((_ROWS, h), lambda i: (i, 0)),
            pl.BlockSpec((_ROWS, 1), lambda i: (i, 0)),
            pl.BlockSpec((1, h), lambda i: (0, 0)),
            pl.BlockSpec((_ROWS, 1), lambda i: (i, 0)),
            pl.BlockSpec((h, emb_d), lambda i: (0, 0)),
            pl.BlockSpec((1, emb_d), lambda i: (0, 0)),
            pl.BlockSpec((emb_d, out_d), lambda i: (0, 0)),
            pl.BlockSpec((1, out_d), lambda i: (0, 0)),
        ],
        out_specs=[
            pl.BlockSpec((n_groups, emb_d), lambda i: (0, 0)),
            pl.BlockSpec((n_groups, out_d), lambda i: (0, 0)),
        ],
        out_shape=[
            jax.ShapeDtypeStruct((n_groups, emb_d), jnp.float32),
            jax.ShapeDtypeStruct((n_groups, out_d), jnp.float32),
        ],
        scratch_shapes=[
            pltpu.VMEM((n_groups, h), jnp.float32),
            pltpu.VMEM((n_groups, 1), jnp.float32),
        ],
        name="tc_head",
    )(p, hs, dinv, b2.reshape(1, h), batchf, fw1, fb1.reshape(1, emb_d),
      fw2, fb2.reshape(1, out_d))


def kernel(x, edge_index, batch, W1, b1, W2, b2, fcW1, fcb1, fcW2, fcb2):
    n = x.shape[0]
    n_groups = 64
    src = edge_index[0]
    dst = edge_index[1]

    degp = _sc_degree(dst, n)
    hs1, dinv = _tc_scale1(degp, x, W1, n)
    p1 = _sc_scatter(src, dst, hs1, n)
    hs2 = _tc_layer2(p1, hs1, dinv, b1, W2, n)
    p2 = _sc_scatter(src, dst, hs2, n)
    batchf = batch.astype(jnp.float32).reshape(n, 1)
    emb, out = _tc_head(p2, hs2, dinv, b2, batchf, fcW1, fcb1, fcW2, fcb2,
                        n, n_groups)
    return (emb, out)


# speedup vs baseline: 16.5816x; 16.5816x over previous
import functools

import jax
import jax.numpy as jnp
from jax import lax
from jax.experimental import pallas as pl
from jax.experimental.pallas import tpu as pltpu
from jax.experimental.pallas import tpu_sc as plsc

_NC = 2
_NS = 16
_NW = _NC * _NS
_CHUNK = 128
_DEGW = 16


def _deg_kernel_body(dst_hbm, degp_hbm, idx_v, deg_v, *, n_pad, n_chunks):
    cid = lax.axis_index("c")
    sid = lax.axis_index("s")
    wid = sid * _NC + cid
    zeros16 = jnp.zeros((16,), jnp.float32)
    ones16 = jnp.ones((16,), jnp.float32)

    def zstep(i, carry):
        deg_v[pl.ds(i * 16, 16)] = zeros16
        return carry

    lax.fori_loop(0, n_pad // 16, zstep, 0)
    n_iters = (n_chunks + _NW - 1) // _NW

    def step(i, carry):
        c = wid + i * _NW

        @pl.when(c < n_chunks)
        def _():
            pltpu.sync_copy(dst_hbm.at[pl.ds(c * _CHUNK, _CHUNK)], idx_v)
            for j in range(_CHUNK // 16):
                iv = idx_v[pl.ds(j * 16, 16)]
                plsc.addupdate_scatter(deg_v, [iv], ones16)

        return carry

    lax.fori_loop(0, n_iters, step, 0)
    pltpu.sync_copy(deg_v, degp_hbm.at[wid])


def _scatter_kernel_body(src_hbm, dst_hbm, hs_hbm, zeros_hbm, part_hbm,
                         sidx_v, didx_v, rows_v, acc_shared, *, n_pad,
                         n_chunks):
    cid = lax.axis_index("c")
    sid = lax.axis_index("s")
    wid = sid * _NC + cid
    rows_per_sub = n_pad // _NS
    row0 = sid * rows_per_sub
    pltpu.sync_copy(zeros_hbm.at[pl.ds(row0, rows_per_sub)],
                    acc_shared.at[pl.ds(row0, rows_per_sub)])
    plsc.subcore_barrier()
    n_iters = (n_chunks + _NW - 1) // _NW

    def step(i, carry):
        c = wid + i * _NW

        @pl.when(c < n_chunks)
        def _():
            off = c * _CHUNK
            pltpu.sync_copy(src_hbm.at[pl.ds(off, _CHUNK)], sidx_v)
            pltpu.sync_copy(dst_hbm.at[pl.ds(off, _CHUNK)], didx_v)
            pltpu.sync_copy(hs_hbm.at[sidx_v], rows_v)
            pltpu.sync_copy(rows_v, acc_shared.at[didx_v], add=True)

        return carry

    lax.fori_loop(0, n_iters, step, 0)
    plsc.subcore_barrier()
    pltpu.sync_copy(acc_shared.at[pl.ds(row0, rows_per_sub)],
                    part_hbm.at[cid, pl.ds(row0, rows_per_sub)])


def _sc_degree(dst, n_pad):
    e = dst.shape[0]
    n_chunks = e // _CHUNK
    mesh = plsc.VectorSubcoreMesh(core_axis_name="c", subcore_axis_name="s")
    body = functools.partial(_deg_kernel_body, n_pad=n_pad,
                             n_chunks=n_chunks)
    return pl.kernel(
        body,
        out_type=jax.ShapeDtypeStruct((_NW, n_pad), jnp.float32),
        mesh=mesh,
        scratch_types=[
            pltpu.VMEM((_CHUNK,), jnp.int32),
            pltpu.VMEM((n_pad,), jnp.float32),
        ],
        compiler_params=pltpu.CompilerParams(needs_layout_passes=False),
        name="sc_degree",
    )(dst)


def _sc_scatter(src, dst, hs, n_pad):
    e = src.shape[0]
    n_chunks = e // _CHUNK
    d = hs.shape[1]
    zeros = jnp.zeros((n_pad, d), jnp.float32)
    mesh = plsc.VectorSubcoreMesh(core_axis_name="c", subcore_axis_name="s")
    body = functools.partial(_scatter_kernel_body, n_pad=n_pad,
                             n_chunks=n_chunks)
    return pl.kernel(
        body,
        out_type=jax.ShapeDtypeStruct((_NC, n_pad, d), jnp.float32),
        mesh=mesh,
        scratch_types=[
            pltpu.VMEM((_CHUNK,), jnp.int32),
            pltpu.VMEM((_CHUNK,), jnp.int32),
            pltpu.VMEM((_CHUNK, d), jnp.float32),
            pltpu.VMEM_SHARED((n_pad, d), jnp.float32),
        ],
        name="sc_scatter",
    )(src, dst, hs, zeros)


_ROWS = 2000


def _tc_scale1_body(degp_ref, x_ref, w_ref, hs_ref, dinv_ref):
    deg = jnp.sum(degp_ref[...], axis=1, keepdims=True)
    dinv = lax.rsqrt(jnp.maximum(deg + 1.0, 1.0))
    h = jnp.dot(x_ref[...], w_ref[...], preferred_element_type=jnp.float32)
    hs_ref[...] = h * dinv
    dinv_ref[...] = dinv


def _tc_scale1(degp, x, w1, n_nodes):
    d, h = w1.shape
    grid = (n_nodes // _ROWS,)
    return pl.pallas_call(
        _tc_scale1_body,
        grid=grid,
        in_specs=[
            pl.BlockSpec((_ROWS, _NW), lambda i: (i, 0)),
            pl.BlockSpec((_ROWS, d), lambda i: (i, 0)),
            pl.BlockSpec((d, h), lambda i: (0, 0)),
        ],
        out_specs=[
            pl.BlockSpec((_ROWS, h), lambda i: (i, 0)),
            pl.BlockSpec((_ROWS, 1), lambda i: (i, 0)),
        ],
        out_shape=[
            jax.ShapeDtypeStruct((n_nodes, h), jnp.float32),
            jax.ShapeDtypeStruct((n_nodes, 1), jnp.float32),
        ],
        name="tc_scale1",
    )(degp, x, w1)


def _tc_layer2_body(p_ref, hs_ref, dinv_ref, b_ref, w_ref, out_ref):
    dinv = dinv_ref[...]
    agg = (p_ref[0] + p_ref[1] + hs_ref[...]) * dinv
    h1 = jnp.maximum(agg + b_ref[...], 0.0)
    out_ref[...] = jnp.dot(h1, w_ref[...],
                           preferred_element_type=jnp.float32) * dinv


def _tc_layer2(p, hs, dinv, b1, w2, n_nodes):
    h, h2 = w2.shape
    grid = (n_nodes // _ROWS,)
    return pl.pallas_call(
        _tc_layer2_body,
        grid=grid,
        in_specs=[
            pl.BlockSpec((_NC, _ROWS, h), lambda i: (0, i, 0)),
            pl.BlockSpec((_ROWS, h), lambda i: (i, 0)),
            pl.BlockSpec((_ROWS, 1), lambda i: (i, 0)),
            pl.BlockSpec((1, h2), lambda i: (0, 0)),
            pl.BlockSpec((h, h2), lambda i: (0, 0)),
        ],
        out_specs=pl.BlockSpec((_ROWS, h2), lambda i: (i, 0)),
        out_shape=jax.ShapeDtypeStruct((n_nodes, h2), jnp.float32),
        name="tc_layer2",
    )(p, hs, dinv, b1.reshape(1, h2), w2)


def _tc_head_body(p_ref, hs_ref, dinv_ref, b_ref, batch_ref, fw1_ref,
                  fb1_ref, fw2_ref, fb2_ref, emb_ref, out_ref,
                  sums_ref, cnt_ref, *, n_groups, n_blocks):
    i = pl.program_id(0)
    agg = (p_ref[0] + p_ref[1] + hs_ref[...]) * dinv_ref[...]
    h2 = jnp.maximum(agg + b_ref[...], 0.0)
    gids = lax.broadcasted_iota(jnp.int32, (h2.shape[0], n_groups),
                                1).astype(jnp.float32)
    oh = (batch_ref[...] == gids).astype(jnp.float32)
    blk_sums = lax.dot_general(oh, h2, (((0,), (0,)), ((), ())),
                               preferred_element_type=jnp.float32)
    blk_cnt = jnp.sum(oh, axis=0)[:, None]

    @pl.when(i == 0)
    def _():
        sums_ref[...] = blk_sums
        cnt_ref[...] = blk_cnt

    @pl.when(i > 0)
    def _():
        sums_ref[...] += blk_sums
        cnt_ref[...] += blk_cnt

    @pl.when(i == n_blocks - 1)
    def _():
        pooled = sums_ref[...] / jnp.maximum(cnt_ref[...], 1.0)
        emb = jnp.maximum(
            jnp.dot(pooled, fw1_ref[...],
                    preferred_element_type=jnp.float32) + fb1_ref[...], 0.0)
        emb_ref[...] = emb
        out_ref[...] = jnp.dot(emb, fw2_ref[...],
                               preferred_element_type=jnp.float32) + fb2_ref[...]


def _tc_head(p, hs, dinv, b2, batchf, fw1, fb1, fw2, fb2, n_nodes, n_groups):
    h = hs.shape[1]
    emb_d = fw1.shape[1]
    out_d = fw2.shape[1]
    n_blocks = n_nodes // _ROWS
    body = functools.partial(_tc_head_body, n_groups=n_groups,
                             n_blocks=n_blocks)
    return pl.pallas_call(
        body,
        grid=(n_blocks,),
        in_specs=[
            pl.BlockSpec((_NC, _ROWS, h), lambda i: (0, i, 0)),
            pl.BlockSpec((_ROWS, h), lambda i: (i, 0)),
            pl.BlockSpec((_ROWS, 1), lambda i: (i, 0)),
            pl.BlockSpec((1, h), lambda i: (0, 0)),
            pl.BlockSpec((_ROWS, 1), lambda i: (i, 0)),
            pl.BlockSpec((h, emb_d), lambda i: (0, 0)),
            pl.BlockSpec((1, emb_d), lambda i: (0, 0)),
            pl.BlockSpec((emb_d, out_d), lambda i: (0, 0)),
            pl.BlockSpec((1, out_d), lambda i: (0, 0)),
        ],
        out_specs=[
            pl.BlockSpec((n_groups, emb_d), lambda i: (0, 0)),
            pl.BlockSpec((n_groups, out_d), lambda i: (0, 0)),
        ],
        out_shape=[
            jax.ShapeDtypeStruct((n_groups, emb_d), jnp.float32),
            jax.ShapeDtypeStruct((n_groups, out_d), jnp.float32),
        ],
        scratch_shapes=[
            pltpu.VMEM((n_groups, h), jnp.float32),
            pltpu.VMEM((n_groups, 1), jnp.float32),
        ],
        name="tc_head",
    )(p, hs, dinv, b2.reshape(1, h), batchf, fw1, fb1.reshape(1, emb_d),
      fw2, fb2.reshape(1, out_d))


def kernel(x, edge_index, batch, W1, b1, W2, b2, fcW1, fcb1, fcW2, fcb2):
    n = x.shape[0]
    n_pad = ((n + 8 * _NS - 1) // (8 * _NS)) * (8 * _NS)
    n_groups = 64
    src = edge_index[0]
    dst = edge_index[1]

    degp = _sc_degree(dst, n_pad).T
    hs1, dinv = _tc_scale1(degp, x, W1, n)
    p1 = _sc_scatter(src, dst, hs1, n_pad)
    hs2 = _tc_layer2(p1, hs1, dinv, b1, W2, n)
    p2 = _sc_scatter(src, dst, hs2, n_pad)
    batchf = batch.astype(jnp.float32).reshape(n, 1)
    emb, out = _tc_head(p2, hs2, dinv, b2, batchf, fcW1, fcb1, fcW2, fcb2,
                        n, n_groups)
    return (emb, out)
